# Initial kernel scaffold; baseline (speedup 1.0000x reference)
import jax
import jax.numpy as jnp
from jax.experimental import pallas as pl


def kernel(x, edge_index, Wl1, bl1, Wr1, Wl2, bl2, Wr2, Wl3, bl3, Wr3):
    raise NotImplementedError("write your pallas kernel here")


# speedup vs baseline: 2.3406x; 2.3406x over previous
import jax
import jax.numpy as jnp
from jax import lax
from jax.experimental import pallas as pl
from jax.experimental.pallas import tpu as pltpu
from jax.experimental.pallas import tpu_sc as plsc

N = 10000
D = 256
DH = 128
NC = 2
NS = 16
LANES = 16
CHUNK = 128
NACC = 10112
ZR = NACC // NS
ZFULL = ZR // CHUNK
ZREM = ZR % CHUNK


def _mesh():
    return plsc.VectorSubcoreMesh(core_axis_name="c", subcore_axis_name="s",
                                  num_cores=NC, num_subcores=NS)


def _sc_agg(e_pad):
    n_chunks = e_pad // (NS * CHUNK)
    ept = n_chunks * CHUNK

    def body(tab, src2, dstp, out, sidx, didx, rows, acc, sem):
        c = lax.axis_index("c")
        s = lax.axis_index("s")
        zv = jnp.zeros((LANES,), jnp.float32)

        def zrow(i, carry):
            for j in range(DH // LANES):
                rows[i, pl.ds(j * LANES, LANES)] = zv
            return carry
        lax.fori_loop(0, CHUNK, zrow, 0)
        for k in range(ZFULL):
            pltpu.sync_copy(rows, acc.at[pl.ds(s * ZR + k * CHUNK, CHUNK)])
        if ZREM:
            pltpu.sync_copy(rows.at[pl.ds(0, ZREM)],
                            acc.at[pl.ds(s * ZR + ZFULL * CHUNK, ZREM)])
        plsc.subcore_barrier()

        base = s * ept

        def step(i, carry):
            off = base + i * CHUNK
            pltpu.sync_copy(src2.at[c, pl.ds(off, CHUNK)], sidx)
            pltpu.sync_copy(dstp.at[pl.ds(off, CHUNK)], didx)
            pltpu.async_copy(tab.at[sidx], rows, sem).wait()
            pltpu.sync_copy(rows, acc.at[didx], add=True)
            return carry
        lax.fori_loop(0, n_chunks, step, 0)
        plsc.subcore_barrier()

        pltpu.sync_copy(acc.at[pl.ds(s * ZR, ZR)], out.at[c, pl.ds(s * ZR, ZR)])

    return pl.kernel(
        body,
        out_type=jax.ShapeDtypeStruct((NC, NACC, DH), jnp.float32),
        mesh=_mesh(),
        scratch_types=(
            pltpu.VMEM((CHUNK,), jnp.int32),
            pltpu.VMEM((CHUNK,), jnp.int32),
            pltpu.VMEM((CHUNK, DH), jnp.float32),
            pltpu.VMEM_SHARED((NACC, DH), jnp.float32),
            pltpu.SemaphoreType.DMA,
        ))


def _sc_deg(e_pad):
    n_chunks = e_pad // (NC * NS * CHUNK)
    ept = n_chunks * CHUNK

    def body(dstp, deg_out, didx, ones, zbuf, deg_acc):
        c = lax.axis_index("c")
        s = lax.axis_index("s")
        zv = jnp.zeros((LANES,), jnp.float32)
        ov = jnp.ones((LANES,), jnp.float32)

        def fill(i, carry):
            for j in range(DH // LANES):
                ones[i, pl.ds(j * LANES, LANES)] = ov
            return carry
        lax.fori_loop(0, CHUNK, fill, 0)

        def zfill(i, carry):
            for j in range(DH // LANES):
                zbuf[i, pl.ds(j * LANES, LANES)] = zv
            return carry
        lax.fori_loop(0, 8, zfill, 0)

        def zero(k, carry):
            pltpu.sync_copy(zbuf, deg_acc.at[pl.ds(s * ZR + k * 8, 8)])
            return carry
        lax.fori_loop(0, ZR // 8, zero, 0)
        plsc.subcore_barrier()

        base = (c * NS + s) * ept

        def step(i, carry):
            off = base + i * CHUNK
            pltpu.sync_copy(dstp.at[pl.ds(off, CHUNK)], didx)
            pltpu.sync_copy(ones, deg_acc.at[didx], add=True)
            return carry
        lax.fori_loop(0, n_chunks, step, 0)
        plsc.subcore_barrier()

        pltpu.sync_copy(deg_acc.at[pl.ds(s * ZR, ZR)],
                        deg_out.at[c, pl.ds(s * ZR, ZR)])

    return pl.kernel(
        body,
        out_type=jax.ShapeDtypeStruct((NC, NACC, DH), jnp.float32),
        mesh=_mesh(),
        scratch_types=(
            pltpu.VMEM((CHUNK,), jnp.int32),
            pltpu.VMEM((CHUNK, DH), jnp.float32),
            pltpu.VMEM((8, DH), jnp.float32),
            pltpu.VMEM_SHARED((NACC, DH), jnp.float32),
        ))


_DN = (((1,), (1,)), ((), ()))


def _tc_layer(agg, deg16, h2, Wl, bl_row, Wr, relu, split_out):
    BM = 1000
    grid = (N // BM,)

    def body(agg_ref, deg_ref, h_ref, wl_ref, bl_ref, wr_ref, out_ref):
        deg = deg_ref[0, :, 0:1] + deg_ref[1, :, 0:1]
        r = 1.0 / jnp.maximum(deg, 1.0)
        m0 = agg_ref[0] * r
        m1 = agg_ref[1] * r
        wl = wl_ref[...]
        wr = wr_ref[...]
        acc = lax.dot_general(m0, wl[:, :DH], _DN,
                              precision=lax.Precision.HIGHEST,
                              preferred_element_type=jnp.float32)
        acc += lax.dot_general(m1, wl[:, DH:], _DN,
                               precision=lax.Precision.HIGHEST,
                               preferred_element_type=jnp.float32)
        acc += lax.dot_general(h_ref[0], wr[:, :DH], _DN,
                               precision=lax.Precision.HIGHEST,
                               preferred_element_type=jnp.float32)
        acc += lax.dot_general(h_ref[1], wr[:, DH:], _DN,
                               precision=lax.Precision.HIGHEST,
                               preferred_element_type=jnp.float32)
        acc += bl_ref[...]
        if relu:
            acc = jnp.maximum(acc, 0.0)
        if split_out:
            out_ref[0] = acc[:, :DH]
            out_ref[1] = acc[:, DH:]
        else:
            out_ref[...] = acc

    out_shape = (jax.ShapeDtypeStruct((NC, N, DH), jnp.float32) if split_out
                 else jax.ShapeDtypeStruct((N, D), jnp.float32))
    out_spec = (pl.BlockSpec((NC, BM, DH), lambda i: (0, i, 0)) if split_out
                else pl.BlockSpec((BM, D), lambda i: (i, 0)))
    return pl.pallas_call(
        body,
        grid=grid,
        in_specs=[
            pl.BlockSpec((NC, BM, DH), lambda i: (0, i, 0)),
            pl.BlockSpec((NC, BM, DH), lambda i: (0, i, 0)),
            pl.BlockSpec((NC, BM, DH), lambda i: (0, i, 0)),
            pl.BlockSpec((D, D), lambda i: (0, 0)),
            pl.BlockSpec((1, D), lambda i: (0, 0)),
            pl.BlockSpec((D, D), lambda i: (0, 0)),
        ],
        out_specs=out_spec,
        out_shape=out_shape,
    )(agg, deg16, h2, Wl, bl_row, Wr)


def kernel(x, edge_index, Wl1, bl1, Wr1, Wl2, bl2, Wr2, Wl3, bl3, Wr3):
    E = edge_index.shape[1]
    grain = NC * NS * CHUNK
    e_pad = ((E + grain - 1) // grain) * grain
    pad = e_pad - E
    src = edge_index[0]
    dst = edge_index[1]
    src_p = jnp.concatenate([src, jnp.zeros((pad,), jnp.int32)])
    dst_p = jnp.concatenate([dst, jnp.full((pad,), N, jnp.int32)])
    src2 = jnp.stack([src_p, src_p + N])
    x2 = jnp.stack([x[:, :DH], x[:, DH:]])

    sc = _sc_agg(e_pad)

    deg16 = _sc_deg(e_pad)(dst_p)
    agg1 = sc(x2.reshape(2 * N, DH), src2, dst_p)
    h1 = _tc_layer(agg1, deg16, x2, Wl1, bl1.reshape(1, D), Wr1, True, True)
    agg2 = sc(h1.reshape(2 * N, DH), src2, dst_p)
    h2 = _tc_layer(agg2, deg16, h1, Wl2, bl2.reshape(1, D), Wr2, True, True)
    agg3 = sc(h2.reshape(2 * N, DH), src2, dst_p)
    return _tc_layer(agg3, deg16, h2, Wl3, bl3.reshape(1, D), Wr3, False, False)


# speedup vs baseline: 3.0144x; 1.2879x over previous
import jax
import jax.numpy as jnp
from jax import lax
from jax.experimental import pallas as pl
from jax.experimental.pallas import tpu as pltpu
from jax.experimental.pallas import tpu_sc as plsc

N = 10000
D = 256
DH = 128
NC = 2
NS = 16
LANES = 16
CHUNK = 128
NACC = 10112
ZR = NACC // NS
ZFULL = ZR // CHUNK
ZREM = ZR % CHUNK


def _mesh():
    return plsc.VectorSubcoreMesh(core_axis_name="c", subcore_axis_name="s",
                                  num_cores=NC, num_subcores=NS)


def _sc_agg(e_pad):
    n_chunks = e_pad // (NS * CHUNK)
    ept = n_chunks * CHUNK

    assert n_chunks >= 2 and n_chunks % 2 == 0

    def body(tab, src2, dstp, out,
             sidx_a, didx_a, rows_a, sem_a,
             sidx_b, didx_b, rows_b, sem_b, acc):
        c = lax.axis_index("c")
        s = lax.axis_index("s")
        zv = jnp.zeros((LANES,), jnp.float32)

        def zrow(i, carry):
            for j in range(DH // LANES):
                rows_a[i, pl.ds(j * LANES, LANES)] = zv
            return carry
        lax.fori_loop(0, CHUNK, zrow, 0)
        for k in range(ZFULL):
            pltpu.sync_copy(rows_a, acc.at[pl.ds(s * ZR + k * CHUNK, CHUNK)])
        if ZREM:
            pltpu.sync_copy(rows_a.at[pl.ds(0, ZREM)],
                            acc.at[pl.ds(s * ZR + ZFULL * CHUNK, ZREM)])
        plsc.subcore_barrier()

        base = s * ept

        def load_idx(i, sidx, didx):
            off = base + i * CHUNK
            pltpu.sync_copy(src2.at[c, pl.ds(off, CHUNK)], sidx)
            pltpu.sync_copy(dstp.at[pl.ds(off, CHUNK)], didx)

        load_idx(0, sidx_a, didx_a)
        pltpu.async_copy(tab.at[sidx_a], rows_a, sem_a)
        load_idx(1, sidx_b, didx_b)
        pltpu.async_copy(tab.at[sidx_b], rows_b, sem_b)

        def pair(k, carry):
            i0 = 2 * k

            pltpu.make_async_copy(tab.at[sidx_a], rows_a, sem_a).wait()
            pltpu.sync_copy(rows_a, acc.at[didx_a], add=True)

            @pl.when(i0 + 2 < n_chunks)
            def _():
                load_idx(i0 + 2, sidx_a, didx_a)
                pltpu.async_copy(tab.at[sidx_a], rows_a, sem_a)

            pltpu.make_async_copy(tab.at[sidx_b], rows_b, sem_b).wait()
            pltpu.sync_copy(rows_b, acc.at[didx_b], add=True)

            @pl.when(i0 + 3 < n_chunks)
            def _():
                load_idx(i0 + 3, sidx_b, didx_b)
                pltpu.async_copy(tab.at[sidx_b], rows_b, sem_b)
            return carry
        lax.fori_loop(0, n_chunks // 2, pair, 0)
        plsc.subcore_barrier()

        pltpu.sync_copy(acc.at[pl.ds(s * ZR, ZR)], out.at[c, pl.ds(s * ZR, ZR)])

    return pl.kernel(
        body,
        out_type=jax.ShapeDtypeStruct((NC, NACC, DH), jnp.float32),
        mesh=_mesh(),
        scratch_types=(
            pltpu.VMEM((CHUNK,), jnp.int32),
            pltpu.VMEM((CHUNK,), jnp.int32),
            pltpu.VMEM((CHUNK, DH), jnp.float32),
            pltpu.SemaphoreType.DMA,
            pltpu.VMEM((CHUNK,), jnp.int32),
            pltpu.VMEM((CHUNK,), jnp.int32),
            pltpu.VMEM((CHUNK, DH), jnp.float32),
            pltpu.SemaphoreType.DMA,
            pltpu.VMEM_SHARED((NACC, DH), jnp.float32),
        ))


def _sc_deg(e_pad):
    n_chunks = e_pad // (NC * NS * CHUNK)
    ept = n_chunks * CHUNK

    def body(dstp, deg_out, didx, ones, zbuf, deg_acc):
        c = lax.axis_index("c")
        s = lax.axis_index("s")
        zv = jnp.zeros((LANES,), jnp.float32)
        ov = jnp.ones((LANES,), jnp.float32)

        def fill(i, carry):
            for j in range(DH // LANES):
                ones[i, pl.ds(j * LANES, LANES)] = ov
            return carry
        lax.fori_loop(0, CHUNK, fill, 0)

        def zfill(i, carry):
            for j in range(DH // LANES):
                zbuf[i, pl.ds(j * LANES, LANES)] = zv
            return carry
        lax.fori_loop(0, 8, zfill, 0)

        def zero(k, carry):
            pltpu.sync_copy(zbuf, deg_acc.at[pl.ds(s * ZR + k * 8, 8)])
            return carry
        lax.fori_loop(0, ZR // 8, zero, 0)
        plsc.subcore_barrier()

        base = (c * NS + s) * ept

        def step(i, carry):
            off = base + i * CHUNK
            pltpu.sync_copy(dstp.at[pl.ds(off, CHUNK)], didx)
            pltpu.sync_copy(ones, deg_acc.at[didx], add=True)
            return carry
        lax.fori_loop(0, n_chunks, step, 0)
        plsc.subcore_barrier()

        pltpu.sync_copy(deg_acc.at[pl.ds(s * ZR, ZR)],
                        deg_out.at[c, pl.ds(s * ZR, ZR)])

    return pl.kernel(
        body,
        out_type=jax.ShapeDtypeStruct((NC, NACC, DH), jnp.float32),
        mesh=_mesh(),
        scratch_types=(
            pltpu.VMEM((CHUNK,), jnp.int32),
            pltpu.VMEM((CHUNK, DH), jnp.float32),
            pltpu.VMEM((8, DH), jnp.float32),
            pltpu.VMEM_SHARED((NACC, DH), jnp.float32),
        ))


_DN = (((1,), (1,)), ((), ()))


def _tc_layer(agg, deg16, h2, Wl, bl_row, Wr, relu, split_out):
    BM = 1000
    grid = (N // BM,)

    def body(agg_ref, deg_ref, h_ref, wl_ref, bl_ref, wr_ref, out_ref):
        deg = deg_ref[0, :, 0:1] + deg_ref[1, :, 0:1]
        r = 1.0 / jnp.maximum(deg, 1.0)
        m0 = agg_ref[0] * r
        m1 = agg_ref[1] * r
        wl = wl_ref[...]
        wr = wr_ref[...]
        acc = lax.dot_general(m0, wl[:, :DH], _DN,
                              precision=lax.Precision.HIGHEST,
                              preferred_element_type=jnp.float32)
        acc += lax.dot_general(m1, wl[:, DH:], _DN,
                               precision=lax.Precision.HIGHEST,
                               preferred_element_type=jnp.float32)
        acc += lax.dot_general(h_ref[0], wr[:, :DH], _DN,
                               precision=lax.Precision.HIGHEST,
                               preferred_element_type=jnp.float32)
        acc += lax.dot_general(h_ref[1], wr[:, DH:], _DN,
                               precision=lax.Precision.HIGHEST,
                               preferred_element_type=jnp.float32)
        acc += bl_ref[...]
        if relu:
            acc = jnp.maximum(acc, 0.0)
        if split_out:
            out_ref[0] = acc[:, :DH]
            out_ref[1] = acc[:, DH:]
        else:
            out_ref[...] = acc

    out_shape = (jax.ShapeDtypeStruct((NC, N, DH), jnp.float32) if split_out
                 else jax.ShapeDtypeStruct((N, D), jnp.float32))
    out_spec = (pl.BlockSpec((NC, BM, DH), lambda i: (0, i, 0)) if split_out
                else pl.BlockSpec((BM, D), lambda i: (i, 0)))
    return pl.pallas_call(
        body,
        grid=grid,
        in_specs=[
            pl.BlockSpec((NC, BM, DH), lambda i: (0, i, 0)),
            pl.BlockSpec((NC, BM, DH), lambda i: (0, i, 0)),
            pl.BlockSpec((NC, BM, DH), lambda i: (0, i, 0)),
            pl.BlockSpec((D, D), lambda i: (0, 0)),
            pl.BlockSpec((1, D), lambda i: (0, 0)),
            pl.BlockSpec((D, D), lambda i: (0, 0)),
        ],
        out_specs=out_spec,
        out_shape=out_shape,
    )(agg, deg16, h2, Wl, bl_row, Wr)


def kernel(x, edge_index, Wl1, bl1, Wr1, Wl2, bl2, Wr2, Wl3, bl3, Wr3):
    E = edge_index.shape[1]
    grain = NC * NS * CHUNK
    e_pad = ((E + grain - 1) // grain) * grain
    pad = e_pad - E
    src = edge_index[0]
    dst = edge_index[1]
    src_p = jnp.concatenate([src, jnp.zeros((pad,), jnp.int32)])
    dst_p = jnp.concatenate([dst, jnp.full((pad,), N, jnp.int32)])
    src2 = jnp.stack([src_p, src_p + N])
    x2 = jnp.stack([x[:, :DH], x[:, DH:]])

    sc = _sc_agg(e_pad)

    deg16 = _sc_deg(e_pad)(dst_p)
    agg1 = sc(x2.reshape(2 * N, DH), src2, dst_p)
    h1 = _tc_layer(agg1, deg16, x2, Wl1, bl1.reshape(1, D), Wr1, True, True)
    agg2 = sc(h1.reshape(2 * N, DH), src2, dst_p)
    h2 = _tc_layer(agg2, deg16, h1, Wl2, bl2.reshape(1, D), Wr2, True, True)
    agg3 = sc(h2.reshape(2 * N, DH), src2, dst_p)
    return _tc_layer(agg3, deg16, h2, Wl3, bl3.reshape(1, D), Wr3, False, False)


# speedup vs baseline: 3.0225x; 1.0027x over previous
import jax
import jax.numpy as jnp
from jax import lax
from jax.experimental import pallas as pl
from jax.experimental.pallas import tpu as pltpu
from jax.experimental.pallas import tpu_sc as plsc

N = 10000
D = 256
DH = 128
NC = 2
NS = 16
LANES = 16
CHUNK = 128
NACC = 10112
ZR = NACC // NS
ZFULL = ZR // CHUNK
ZREM = ZR % CHUNK


def _mesh():
    return plsc.VectorSubcoreMesh(core_axis_name="c", subcore_axis_name="s",
                                  num_cores=NC, num_subcores=NS)


def _sc_agg(e_pad):
    n_chunks = e_pad // (NS * CHUNK)
    ept = n_chunks * CHUNK

    assert n_chunks >= 2 and n_chunks % 2 == 0

    def body(tab, src2, dstp, out,
             sidx_a, didx_a, rows_a, sem_a, sem_sa,
             sidx_b, didx_b, rows_b, sem_b, sem_sb, acc):
        c = lax.axis_index("c")
        s = lax.axis_index("s")
        zv = jnp.zeros((LANES,), jnp.float32)

        def zrow(i, carry):
            for j in range(DH // LANES):
                rows_a[i, pl.ds(j * LANES, LANES)] = zv
            return carry
        lax.fori_loop(0, CHUNK, zrow, 0)
        for k in range(ZFULL):
            pltpu.sync_copy(rows_a, acc.at[pl.ds(s * ZR + k * CHUNK, CHUNK)])
        if ZREM:
            pltpu.sync_copy(rows_a.at[pl.ds(0, ZREM)],
                            acc.at[pl.ds(s * ZR + ZFULL * CHUNK, ZREM)])
        plsc.subcore_barrier()

        base = s * ept

        def load_idx(i, sidx, didx, sem):
            off = base + i * CHUNK
            pltpu.async_copy(src2.at[c, pl.ds(off, CHUNK)], sidx, sem)
            pltpu.async_copy(dstp.at[pl.ds(off, CHUNK)], didx, sem)

        def wait_idx(sidx, didx, sem):
            pltpu.make_async_copy(src2.at[c, pl.ds(0, CHUNK)], sidx, sem).wait()
            pltpu.make_async_copy(dstp.at[pl.ds(0, CHUNK)], didx, sem).wait()

        def wait_scat(rows, didx, sem):
            pltpu.make_async_copy(rows, acc.at[didx], sem).wait()

        load_idx(0, sidx_a, didx_a, sem_a)
        wait_idx(sidx_a, didx_a, sem_a)
        pltpu.async_copy(tab.at[sidx_a], rows_a, sem_a)
        load_idx(1, sidx_b, didx_b, sem_b)
        wait_idx(sidx_b, didx_b, sem_b)
        pltpu.async_copy(tab.at[sidx_b], rows_b, sem_b)

        def pair(k, carry):
            i0 = 2 * k

            pltpu.make_async_copy(tab.at[sidx_a], rows_a, sem_a).wait()
            pltpu.async_copy(rows_a, acc.at[didx_a], sem_sa, add=True)

            pltpu.make_async_copy(tab.at[sidx_b], rows_b, sem_b).wait()
            pltpu.async_copy(rows_b, acc.at[didx_b], sem_sb, add=True)

            @pl.when(i0 + 2 < n_chunks)
            def _():
                wait_scat(rows_a, didx_a, sem_sa)
                load_idx(i0 + 2, sidx_a, didx_a, sem_a)
                wait_idx(sidx_a, didx_a, sem_a)
                pltpu.async_copy(tab.at[sidx_a], rows_a, sem_a)

            @pl.when(i0 + 3 < n_chunks)
            def _():
                wait_scat(rows_b, didx_b, sem_sb)
                load_idx(i0 + 3, sidx_b, didx_b, sem_b)
                wait_idx(sidx_b, didx_b, sem_b)
                pltpu.async_copy(tab.at[sidx_b], rows_b, sem_b)
            return carry
        lax.fori_loop(0, n_chunks // 2, pair, 0)
        wait_scat(rows_a, didx_a, sem_sa)
        wait_scat(rows_b, didx_b, sem_sb)
        plsc.subcore_barrier()

        pltpu.sync_copy(acc.at[pl.ds(s * ZR, ZR)], out.at[c, pl.ds(s * ZR, ZR)])

    return pl.kernel(
        body,
        out_type=jax.ShapeDtypeStruct((NC, NACC, DH), jnp.float32),
        mesh=_mesh(),
        scratch_types=(
            pltpu.VMEM((CHUNK,), jnp.int32),
            pltpu.VMEM((CHUNK,), jnp.int32),
            pltpu.VMEM((CHUNK, DH), jnp.float32),
            pltpu.SemaphoreType.DMA,
            pltpu.SemaphoreType.DMA,
            pltpu.VMEM((CHUNK,), jnp.int32),
            pltpu.VMEM((CHUNK,), jnp.int32),
            pltpu.VMEM((CHUNK, DH), jnp.float32),
            pltpu.SemaphoreType.DMA,
            pltpu.SemaphoreType.DMA,
            pltpu.VMEM_SHARED((NACC, DH), jnp.float32),
        ))


def _sc_deg(e_pad):
    n_chunks = e_pad // (NC * NS * CHUNK)
    ept = n_chunks * CHUNK

    def body(dstp, deg_out, didx, ones, zbuf, deg_acc):
        c = lax.axis_index("c")
        s = lax.axis_index("s")
        zv = jnp.zeros((LANES,), jnp.float32)
        ov = jnp.ones((LANES,), jnp.float32)

        def fill(i, carry):
            for j in range(DH // LANES):
                ones[i, pl.ds(j * LANES, LANES)] = ov
            return carry
        lax.fori_loop(0, CHUNK, fill, 0)

        def zfill(i, carry):
            for j in range(DH // LANES):
                zbuf[i, pl.ds(j * LANES, LANES)] = zv
            return carry
        lax.fori_loop(0, 8, zfill, 0)

        def zero(k, carry):
            pltpu.sync_copy(zbuf, deg_acc.at[pl.ds(s * ZR + k * 8, 8)])
            return carry
        lax.fori_loop(0, ZR // 8, zero, 0)
        plsc.subcore_barrier()

        base = (c * NS + s) * ept

        def step(i, carry):
            off = base + i * CHUNK
            pltpu.sync_copy(dstp.at[pl.ds(off, CHUNK)], didx)
            pltpu.sync_copy(ones, deg_acc.at[didx], add=True)
            return carry
        lax.fori_loop(0, n_chunks, step, 0)
        plsc.subcore_barrier()

        pltpu.sync_copy(deg_acc.at[pl.ds(s * ZR, ZR)],
                        deg_out.at[c, pl.ds(s * ZR, ZR)])

    return pl.kernel(
        body,
        out_type=jax.ShapeDtypeStruct((NC, NACC, DH), jnp.float32),
        mesh=_mesh(),
        scratch_types=(
            pltpu.VMEM((CHUNK,), jnp.int32),
            pltpu.VMEM((CHUNK, DH), jnp.float32),
            pltpu.VMEM((8, DH), jnp.float32),
            pltpu.VMEM_SHARED((NACC, DH), jnp.float32),
        ))


_DN = (((1,), (1,)), ((), ()))


def _tc_layer(agg, deg16, h2, Wl, bl_row, Wr, relu, split_out):
    BM = 1000
    grid = (N // BM,)

    def body(agg_ref, deg_ref, h_ref, wl_ref, bl_ref, wr_ref, out_ref):
        deg = deg_ref[0, :, 0:1] + deg_ref[1, :, 0:1]
        r = 1.0 / jnp.maximum(deg, 1.0)
        m0 = agg_ref[0] * r
        m1 = agg_ref[1] * r
        wl = wl_ref[...]
        wr = wr_ref[...]
        acc = lax.dot_general(m0, wl[:, :DH], _DN,
                              precision=lax.Precision.HIGHEST,
                              preferred_element_type=jnp.float32)
        acc += lax.dot_general(m1, wl[:, DH:], _DN,
                               precision=lax.Precision.HIGHEST,
                               preferred_element_type=jnp.float32)
        acc += lax.dot_general(h_ref[0], wr[:, :DH], _DN,
                               precision=lax.Precision.HIGHEST,
                               preferred_element_type=jnp.float32)
        acc += lax.dot_general(h_ref[1], wr[:, DH:], _DN,
                               precision=lax.Precision.HIGHEST,
                               preferred_element_type=jnp.float32)
        acc += bl_ref[...]
        if relu:
            acc = jnp.maximum(acc, 0.0)
        if split_out:
            out_ref[0] = acc[:, :DH]
            out_ref[1] = acc[:, DH:]
        else:
            out_ref[...] = acc

    out_shape = (jax.ShapeDtypeStruct((NC, N, DH), jnp.float32) if split_out
                 else jax.ShapeDtypeStruct((N, D), jnp.float32))
    out_spec = (pl.BlockSpec((NC, BM, DH), lambda i: (0, i, 0)) if split_out
                else pl.BlockSpec((BM, D), lambda i: (i, 0)))
    return pl.pallas_call(
        body,
        grid=grid,
        in_specs=[
            pl.BlockSpec((NC, BM, DH), lambda i: (0, i, 0)),
            pl.BlockSpec((NC, BM, DH), lambda i: (0, i, 0)),
            pl.BlockSpec((NC, BM, DH), lambda i: (0, i, 0)),
            pl.BlockSpec((D, D), lambda i: (0, 0)),
            pl.BlockSpec((1, D), lambda i: (0, 0)),
            pl.BlockSpec((D, D), lambda i: (0, 0)),
        ],
        out_specs=out_spec,
        out_shape=out_shape,
    )(agg, deg16, h2, Wl, bl_row, Wr)


def kernel(x, edge_index, Wl1, bl1, Wr1, Wl2, bl2, Wr2, Wl3, bl3, Wr3):
    E = edge_index.shape[1]
    grain = NC * NS * CHUNK
    e_pad = ((E + grain - 1) // grain) * grain
    pad = e_pad - E
    src = edge_index[0]
    dst = edge_index[1]
    src_p = jnp.concatenate([src, jnp.zeros((pad,), jnp.int32)])
    dst_p = jnp.concatenate([dst, jnp.full((pad,), N, jnp.int32)])
    src2 = jnp.stack([src_p, src_p + N])
    x2 = jnp.stack([x[:, :DH], x[:, DH:]])

    sc = _sc_agg(e_pad)

    deg16 = _sc_deg(e_pad)(dst_p)
    agg1 = sc(x2.reshape(2 * N, DH), src2, dst_p)
    h1 = _tc_layer(agg1, deg16, x2, Wl1, bl1.reshape(1, D), Wr1, True, True)
    agg2 = sc(h1.reshape(2 * N, DH), src2, dst_p)
    h2 = _tc_layer(agg2, deg16, h1, Wl2, bl2.reshape(1, D), Wr2, True, True)
    agg3 = sc(h2.reshape(2 * N, DH), src2, dst_p)
    return _tc_layer(agg3, deg16, h2, Wl3, bl3.reshape(1, D), Wr3, False, False)
